# Initial kernel scaffold; baseline (speedup 1.0000x reference)
import jax
import jax.numpy as jnp
from jax.experimental import pallas as pl
from jax.experimental.pallas import tpu as pltpu

_C = 128
_TT = 256
_VT = 2048
_NORM_EPS = 1e-12
_LN_EPS = 1e-5


def _f32dot(a, b, dims):
    return jax.lax.dot_general(a, b, (dims, ((), ())),
                               preferred_element_type=jnp.float32)


def _encoder_body(e_ref, w1_ref, b1_ref, w2_ref, b2_ref, g_ref, bb_ref,
                  kp_ref, k_ref):
    e = e_ref[...]
    z = jnp.maximum(
        jnp.dot(e, w1_ref[...], preferred_element_type=jnp.float32)
        + b1_ref[...], 0.0)
    ff = jnp.dot(z, w2_ref[...], preferred_element_type=jnp.float32) \
        + b2_ref[...]
    x = e + ff
    mu = jnp.mean(x, axis=1, keepdims=True)
    xc = x - mu
    var = jnp.mean(xc * xc, axis=1, keepdims=True)
    h = xc * jax.lax.rsqrt(var + _LN_EPS) * g_ref[...] + bb_ref[...]
    k_ref[...] = jnp.dot(h, kp_ref[...], preferred_element_type=jnp.float32)


def _scan_body(ks_ref, rpw_ref, rpb_ref, out_ref, m_ref):
    c = pl.program_id(1)
    nc = pl.num_programs(1)
    k_raw = ks_ref[0, 0]

    row = jax.lax.broadcasted_iota(jnp.int32, (_C, 1), 0)
    valid = jnp.logical_or(c < nc - 1, row < _C - 1)
    km = jnp.where(valid, k_raw, 0.0)

    nrm = jnp.sqrt(jnp.sum(km * km, axis=1, keepdims=True))
    wn = km / jnp.maximum(nrm, _NORM_EPS)

    @pl.when(c == 0)
    def _():
        m_ref[...] = jnp.zeros_like(m_ref)
    m = m_ref[...]

    s = _f32dot(wn, wn, ((1,), (1,)))
    ri = jax.lax.broadcasted_iota(jnp.int32, (_C, _C), 0)
    ci = jax.lax.broadcasted_iota(jnp.int32, (_C, _C), 1)
    a = jnp.where(ri > ci, s, 0.0)
    eye = jnp.where(ri == ci, 1.0, 0.0)

    t = eye - a
    for _ in range(6):
        resid = eye - (t + _f32dot(a, t, ((1,), (0,))))
        t = t + _f32dot(t, resid, ((1,), (0,)))

    rhs = km - _f32dot(wn, m, ((1,), (1,)))
    u = _f32dot(t, rhs, ((1,), (0,)))
    m_new = m + _f32dot(u, wn, ((0,), (0,)))
    m_ref[...] = m_new

    @pl.when(c == nc - 1)
    def _():
        q = k_raw[_C - 1:_C, :]
        r = _f32dot(q, m_new, ((1,), (1,)))
        out_ref[...] = jnp.dot(r, rpw_ref[...],
                               preferred_element_type=jnp.float32) \
            + rpb_ref[...]


def _head_body(rr_ref, w_ref, b_ref, o_ref):
    o_ref[...] = jnp.dot(rr_ref[...], w_ref[...],
                         preferred_element_type=jnp.float32) + b_ref[...]


def kernel(seq, embed, w1, b1, w2, b2, ln_g, ln_b, kp_w, rp_w, rp_b,
           out_w, out_b):
    bsz, slen = seq.shape
    vocab, hdim = embed.shape
    hid2 = w1.shape[1]
    ntok = bsz * slen

    e = embed[jnp.reshape(seq, (-1,))]

    full = lambda shape: pl.BlockSpec(shape, lambda i: (0, 0))
    k_all = pl.pallas_call(
        _encoder_body,
        grid=(ntok // _TT,),
        in_specs=[
            pl.BlockSpec((_TT, hdim), lambda i: (i, 0)),
            full((hdim, hid2)), full((1, hid2)),
            full((hid2, hdim)), full((1, hdim)),
            full((1, hdim)), full((1, hdim)),
            full((hdim, hdim)),
        ],
        out_specs=pl.BlockSpec((_TT, hdim), lambda i: (i, 0)),
        out_shape=jax.ShapeDtypeStruct((ntok, hdim), jnp.float32),
        compiler_params=pltpu.CompilerParams(
            dimension_semantics=("parallel",)),
    )(e, w1, b1.reshape(1, -1), w2, b2.reshape(1, -1),
      ln_g.reshape(1, -1), ln_b.reshape(1, -1), kp_w)

    nc = slen // _C
    ks = k_all.reshape(bsz, nc, _C, hdim)
    rr = pl.pallas_call(
        _scan_body,
        grid=(bsz, nc),
        in_specs=[
            pl.BlockSpec((1, 1, _C, hdim), lambda b, c: (b, c, 0, 0)),
            pl.BlockSpec((hdim, hdim), lambda b, c: (0, 0)),
            pl.BlockSpec((1, hdim), lambda b, c: (0, 0)),
        ],
        out_specs=pl.BlockSpec((1, hdim), lambda b, c: (b, 0)),
        out_shape=jax.ShapeDtypeStruct((bsz, hdim), jnp.float32),
        scratch_shapes=[pltpu.VMEM((hdim, hdim), jnp.float32)],
        compiler_params=pltpu.CompilerParams(
            dimension_semantics=("parallel", "arbitrary")),
    )(ks, rp_w, rp_b.reshape(1, -1))

    out = pl.pallas_call(
        _head_body,
        grid=(vocab // _VT,),
        in_specs=[
            pl.BlockSpec((bsz, hdim), lambda i: (0, 0)),
            pl.BlockSpec((hdim, _VT), lambda i: (0, i)),
            pl.BlockSpec((1, _VT), lambda i: (0, i)),
        ],
        out_specs=pl.BlockSpec((bsz, _VT), lambda i: (0, i)),
        out_shape=jax.ShapeDtypeStruct((bsz, vocab), jnp.float32),
        compiler_params=pltpu.CompilerParams(
            dimension_semantics=("parallel",)),
    )(rr, out_w, out_b.reshape(1, -1))
    return out


# speedup vs baseline: 8.3691x; 8.3691x over previous
import jax
import jax.numpy as jnp
from jax.experimental import pallas as pl
from jax.experimental.pallas import tpu as pltpu

_C = 128
_TT = 256
_VT = 3200
_NORM_EPS = 1e-12
_LN_EPS = 1e-5


def _f32dot(a, b, dims):
    return jax.lax.dot_general(a, b, (dims, ((), ())),
                               preferred_element_type=jnp.float32)


def _encoder_body(e_ref, w1_ref, b1_ref, w2_ref, b2_ref, g_ref, bb_ref,
                  kp_ref, k_ref):
    e = e_ref[...]
    z = jnp.maximum(
        jnp.dot(e, w1_ref[...], preferred_element_type=jnp.float32)
        + b1_ref[...], 0.0)
    ff = jnp.dot(z, w2_ref[...], preferred_element_type=jnp.float32) \
        + b2_ref[...]
    x = e + ff
    mu = jnp.mean(x, axis=1, keepdims=True)
    xc = x - mu
    var = jnp.mean(xc * xc, axis=1, keepdims=True)
    h = xc * jax.lax.rsqrt(var + _LN_EPS) * g_ref[...] + bb_ref[...]
    k_ref[...] = jnp.dot(h, kp_ref[...], preferred_element_type=jnp.float32)


def _scan_body(ks_ref, rpw_ref, rpb_ref, out_ref, m_ref):
    c = pl.program_id(1)
    nc = pl.num_programs(1)
    k_raw = ks_ref[0, 0]

    row = jax.lax.broadcasted_iota(jnp.int32, (_C, 1), 0)
    valid = jnp.logical_or(c < nc - 1, row < _C - 1)
    km = jnp.where(valid, k_raw, 0.0)

    nrm = jnp.sqrt(jnp.sum(km * km, axis=1, keepdims=True))
    wn = km / jnp.maximum(nrm, _NORM_EPS)

    @pl.when(c == 0)
    def _():
        m_ref[...] = jnp.zeros_like(m_ref)
    m = m_ref[...]

    s = _f32dot(wn, wn, ((1,), (1,)))
    ri = jax.lax.broadcasted_iota(jnp.int32, (_C, _C), 0)
    ci = jax.lax.broadcasted_iota(jnp.int32, (_C, _C), 1)
    a = jnp.where(ri > ci, s, 0.0)
    eye = jnp.where(ri == ci, 1.0, 0.0)

    t = eye - a
    for _ in range(6):
        resid = eye - (t + _f32dot(a, t, ((1,), (0,))))
        t = t + _f32dot(t, resid, ((1,), (0,)))

    rhs = km - _f32dot(wn, m, ((1,), (1,)))
    u = _f32dot(t, rhs, ((1,), (0,)))
    m_new = m + _f32dot(u, wn, ((0,), (0,)))
    m_ref[...] = m_new

    @pl.when(c == nc - 1)
    def _():
        q = k_raw[_C - 1:_C, :]
        r = _f32dot(q, m_new, ((1,), (1,)))
        out_ref[0] = jnp.dot(r, rpw_ref[...],
                             preferred_element_type=jnp.float32) \
            + rpb_ref[...]


def _head_body(rr_ref, w_ref, b_ref, o_ref):
    o_ref[...] = jnp.dot(rr_ref[...], w_ref[...],
                         preferred_element_type=jnp.float32) + b_ref[...]


def kernel(seq, embed, w1, b1, w2, b2, ln_g, ln_b, kp_w, rp_w, rp_b,
           out_w, out_b):
    bsz, slen = seq.shape
    vocab, hdim = embed.shape
    hid2 = w1.shape[1]
    ntok = bsz * slen

    e = embed[jnp.reshape(seq, (-1,))]

    full = lambda shape: pl.BlockSpec(shape, lambda i: (0, 0))
    k_all = pl.pallas_call(
        _encoder_body,
        grid=(ntok // _TT,),
        in_specs=[
            pl.BlockSpec((_TT, hdim), lambda i: (i, 0)),
            full((hdim, hid2)), full((1, hid2)),
            full((hid2, hdim)), full((1, hdim)),
            full((1, hdim)), full((1, hdim)),
            full((hdim, hdim)),
        ],
        out_specs=pl.BlockSpec((_TT, hdim), lambda i: (i, 0)),
        out_shape=jax.ShapeDtypeStruct((ntok, hdim), jnp.float32),
        compiler_params=pltpu.CompilerParams(
            dimension_semantics=("parallel",)),
    )(e, w1, b1.reshape(1, -1), w2, b2.reshape(1, -1),
      ln_g.reshape(1, -1), ln_b.reshape(1, -1), kp_w)

    nc = slen // _C
    ks = k_all.reshape(bsz, nc, _C, hdim)
    rr = pl.pallas_call(
        _scan_body,
        grid=(bsz, nc),
        in_specs=[
            pl.BlockSpec((1, 1, _C, hdim), lambda b, c: (b, c, 0, 0)),
            pl.BlockSpec((hdim, hdim), lambda b, c: (0, 0)),
            pl.BlockSpec((1, hdim), lambda b, c: (0, 0)),
        ],
        out_specs=pl.BlockSpec((1, 1, hdim), lambda b, c: (b, 0, 0)),
        out_shape=jax.ShapeDtypeStruct((bsz, 1, hdim), jnp.float32),
        scratch_shapes=[pltpu.VMEM((hdim, hdim), jnp.float32)],
        compiler_params=pltpu.CompilerParams(
            dimension_semantics=("parallel", "arbitrary")),
    )(ks, rp_w, rp_b.reshape(1, -1))
    rr = rr.reshape(bsz, hdim)

    out = pl.pallas_call(
        _head_body,
        grid=(vocab // _VT,),
        in_specs=[
            pl.BlockSpec((bsz, hdim), lambda i: (0, 0)),
            pl.BlockSpec((hdim, _VT), lambda i: (0, i)),
            pl.BlockSpec((1, _VT), lambda i: (0, i)),
        ],
        out_specs=pl.BlockSpec((bsz, _VT), lambda i: (0, i)),
        out_shape=jax.ShapeDtypeStruct((bsz, vocab), jnp.float32),
        compiler_params=pltpu.CompilerParams(
            dimension_semantics=("parallel",)),
    )(rr, out_w, out_b.reshape(1, -1))
    return out


# speedup vs baseline: 9.6015x; 1.1473x over previous
import jax
import jax.numpy as jnp
from jax.experimental import pallas as pl
from jax.experimental.pallas import tpu as pltpu

_C = 128
_G = 8
_TT = 256
_VT = 3200
_NORM_EPS = 1e-12
_LN_EPS = 1e-5


def _f32dot(a, b, dims):
    return jax.lax.dot_general(a, b, (dims, ((), ())),
                               preferred_element_type=jnp.float32)


def _encoder_body(e_ref, w1_ref, b1_ref, w2_ref, b2_ref, g_ref, bb_ref,
                  kp_ref, k_ref):
    e = e_ref[...]
    z = jnp.maximum(
        jnp.dot(e, w1_ref[...], preferred_element_type=jnp.float32)
        + b1_ref[...], 0.0)
    ff = jnp.dot(z, w2_ref[...], preferred_element_type=jnp.float32) \
        + b2_ref[...]
    x = e + ff
    mu = jnp.mean(x, axis=1, keepdims=True)
    xc = x - mu
    var = jnp.mean(xc * xc, axis=1, keepdims=True)
    h = xc * jax.lax.rsqrt(var + _LN_EPS) * g_ref[...] + bb_ref[...]
    k_ref[...] = jnp.dot(h, kp_ref[...], preferred_element_type=jnp.float32)


def _chunk_step(k_raw, valid, m):
    km = jnp.where(valid, k_raw, 0.0)
    nrm = jnp.sqrt(jnp.sum(km * km, axis=1, keepdims=True))
    wn = km / jnp.maximum(nrm, _NORM_EPS)

    s = _f32dot(wn, wn, ((1,), (1,)))
    ri = jax.lax.broadcasted_iota(jnp.int32, (_C, _C), 0)
    ci = jax.lax.broadcasted_iota(jnp.int32, (_C, _C), 1)
    a = jnp.where(ri > ci, s, 0.0)
    eye = jnp.where(ri == ci, 1.0, 0.0)

    t = eye - a
    for _ in range(6):
        resid = eye - (t + _f32dot(a, t, ((1,), (0,))))
        t = t + _f32dot(t, resid, ((1,), (0,)))

    rhs = km - _f32dot(wn, m, ((1,), (1,)))
    u = _f32dot(t, rhs, ((1,), (0,)))
    m_new = m + _f32dot(u, wn, ((0,), (0,)))
    q = k_raw[_C - 1:_C, :]
    return m_new, q


def _scan_body(ks_ref, rpw_ref, rpb_ref, out_ref, m_ref):
    c = pl.program_id(1)
    nc = pl.num_programs(1)

    row = jax.lax.broadcasted_iota(jnp.int32, (_C, 1), 0)
    valid = jnp.logical_or(c < nc - 1, row < _C - 1)

    @pl.when(c == 0)
    def _():
        m_ref[...] = jnp.zeros_like(m_ref)

    rs = []
    for gi in range(_G):
        m_new, q = _chunk_step(ks_ref[0, gi, 0], valid, m_ref[gi])
        m_ref[gi] = m_new
        rs.append(_f32dot(q, m_new, ((1,), (1,))))

    @pl.when(c == nc - 1)
    def _():
        r = jnp.concatenate(rs, axis=0)
        out_ref[0] = jnp.dot(r, rpw_ref[...],
                             preferred_element_type=jnp.float32) \
            + rpb_ref[...]


def _head_body(rr_ref, w_ref, b_ref, o_ref):
    o_ref[...] = jnp.dot(rr_ref[...], w_ref[...],
                         preferred_element_type=jnp.float32) + b_ref[...]


def kernel(seq, embed, w1, b1, w2, b2, ln_g, ln_b, kp_w, rp_w, rp_b,
           out_w, out_b):
    bsz, slen = seq.shape
    vocab, hdim = embed.shape
    hid2 = w1.shape[1]
    ntok = bsz * slen
    ng = bsz // _G

    e = embed[jnp.reshape(seq, (-1,))]

    full = lambda shape: pl.BlockSpec(shape, lambda i: (0, 0))
    k_all = pl.pallas_call(
        _encoder_body,
        grid=(ntok // _TT,),
        in_specs=[
            pl.BlockSpec((_TT, hdim), lambda i: (i, 0)),
            full((hdim, hid2)), full((1, hid2)),
            full((hid2, hdim)), full((1, hdim)),
            full((1, hdim)), full((1, hdim)),
            full((hdim, hdim)),
        ],
        out_specs=pl.BlockSpec((_TT, hdim), lambda i: (i, 0)),
        out_shape=jax.ShapeDtypeStruct((ntok, hdim), jnp.float32),
        compiler_params=pltpu.CompilerParams(
            dimension_semantics=("parallel",)),
    )(e, w1, b1.reshape(1, -1), w2, b2.reshape(1, -1),
      ln_g.reshape(1, -1), ln_b.reshape(1, -1), kp_w)

    nc = slen // _C
    ks = k_all.reshape(ng, _G, nc, _C, hdim)
    rr = pl.pallas_call(
        _scan_body,
        grid=(ng, nc),
        in_specs=[
            pl.BlockSpec((1, _G, 1, _C, hdim),
                         lambda g, c: (g, 0, c, 0, 0)),
            pl.BlockSpec((hdim, hdim), lambda g, c: (0, 0)),
            pl.BlockSpec((1, hdim), lambda g, c: (0, 0)),
        ],
        out_specs=pl.BlockSpec((1, _G, hdim), lambda g, c: (g, 0, 0)),
        out_shape=jax.ShapeDtypeStruct((ng, _G, hdim), jnp.float32),
        scratch_shapes=[pltpu.VMEM((_G, hdim, hdim), jnp.float32)],
        compiler_params=pltpu.CompilerParams(
            dimension_semantics=("parallel", "arbitrary")),
    )(ks, rp_w, rp_b.reshape(1, -1))
    rr = rr.reshape(bsz, hdim)

    out = pl.pallas_call(
        _head_body,
        grid=(vocab // _VT,),
        in_specs=[
            pl.BlockSpec((bsz, hdim), lambda i: (0, 0)),
            pl.BlockSpec((hdim, _VT), lambda i: (0, i)),
            pl.BlockSpec((1, _VT), lambda i: (0, i)),
        ],
        out_specs=pl.BlockSpec((bsz, _VT), lambda i: (0, i)),
        out_shape=jax.ShapeDtypeStruct((bsz, vocab), jnp.float32),
        compiler_params=pltpu.CompilerParams(
            dimension_semantics=("parallel",)),
    )(rr, out_w, out_b.reshape(1, -1))
    return out


# speedup vs baseline: 21.6302x; 2.2528x over previous
import jax
import jax.numpy as jnp
from jax.experimental import pallas as pl
from jax.experimental.pallas import tpu as pltpu

_C = 128
_G = 8
_TT = 256
_VT = 3200
_NORM_EPS = 1e-12
_LN_EPS = 1e-5


def _f32dot(a, b, dims):
    return jax.lax.dot_general(a, b, (dims, ((), ())),
                               preferred_element_type=jnp.float32)


def _bdot(a, b, dims):
    return jax.lax.dot_general(a.astype(jnp.bfloat16), b.astype(jnp.bfloat16),
                               (dims, ((), ())),
                               preferred_element_type=jnp.float32)


def _encoder_body(e_ref, w1_ref, b1_ref, w2_ref, b2_ref, g_ref, bb_ref,
                  kp_ref, k_ref):
    e = e_ref[...]
    z = jnp.maximum(
        jnp.dot(e, w1_ref[...], preferred_element_type=jnp.float32)
        + b1_ref[...], 0.0)
    ff = jnp.dot(z, w2_ref[...], preferred_element_type=jnp.float32) \
        + b2_ref[...]
    x = e + ff
    mu = jnp.mean(x, axis=1, keepdims=True)
    xc = x - mu
    var = jnp.mean(xc * xc, axis=1, keepdims=True)
    h = xc * jax.lax.rsqrt(var + _LN_EPS) * g_ref[...] + bb_ref[...]
    k_ref[...] = jnp.dot(h, kp_ref[...], preferred_element_type=jnp.float32)


def _scan_body(ks_ref, rpw_ref, rpb_ref, out_ref, *m_refs):
    c = pl.program_id(1)
    nc = pl.num_programs(1)

    row = jax.lax.broadcasted_iota(jnp.int32, (_C, 1), 0)
    valid = jnp.logical_or(c < nc - 1, row < _C - 1)

    @pl.when(c == 0)
    def _():
        for m_ref in m_refs:
            m_ref[...] = jnp.zeros_like(m_ref)

    ri = jax.lax.broadcasted_iota(jnp.int32, (_C, _C), 0)
    ci = jax.lax.broadcasted_iota(jnp.int32, (_C, _C), 1)
    eye = jnp.where(ri == ci, 1.0, 0.0)

    g_rng = range(_G)
    k_raws = [ks_ref[0, gi, 0] for gi in g_rng]
    kms = [jnp.where(valid, k, 0.0) for k in k_raws]
    nrms = [jnp.sqrt(jnp.sum(km * km, axis=1, keepdims=True)) for km in kms]
    wns = [km / jnp.maximum(n, _NORM_EPS) for km, n in zip(kms, nrms)]
    wnbs = [wn.astype(jnp.bfloat16) for wn in wns]

    ss = [jax.lax.dot_general(wb, wb, ((((1,), (1,))), ((), ())),
                              preferred_element_type=jnp.float32)
          for wb in wnbs]
    abs_ = [jnp.where(ri > ci, s, 0.0).astype(jnp.bfloat16) for s in ss]

    ts = [eye - ab.astype(jnp.float32) for ab in abs_]
    for _ in range(6):
        tbs = [t.astype(jnp.bfloat16) for t in ts]
        ats = [_bdot(ab, tb, ((1,), (0,))) for ab, tb in zip(abs_, tbs)]
        resids = [(eye - t - at).astype(jnp.bfloat16)
                  for t, at in zip(ts, ats)]
        ts = [t + _bdot(tb, rs_, ((1,), (0,)))
              for t, tb, rs_ in zip(ts, tbs, resids)]

    ms = [m_ref[...] for m_ref in m_refs]
    rhss = [km - _bdot(wb, m, ((1,), (1,)))
            for km, wb, m in zip(kms, wnbs, ms)]
    us = [_bdot(t, rhs, ((1,), (0,))) for t, rhs in zip(ts, rhss)]
    m_news = [m + _bdot(u, wb, ((0,), (0,)))
              for m, u, wb in zip(ms, us, wnbs)]
    rs = []
    for gi in g_rng:
        m_refs[gi][...] = m_news[gi]
        q = k_raws[gi][_C - 1:_C, :]
        rs.append(_f32dot(q, m_news[gi], ((1,), (1,))))

    @pl.when(c == nc - 1)
    def _():
        r = jnp.concatenate(rs, axis=0)
        out_ref[0] = jnp.dot(r, rpw_ref[...],
                             preferred_element_type=jnp.float32) \
            + rpb_ref[...]


def _head_body(rr_ref, w_ref, b_ref, o_ref):
    o_ref[...] = jnp.dot(rr_ref[...], w_ref[...],
                         preferred_element_type=jnp.float32) + b_ref[...]


def kernel(seq, embed, w1, b1, w2, b2, ln_g, ln_b, kp_w, rp_w, rp_b,
           out_w, out_b):
    bsz, slen = seq.shape
    vocab, hdim = embed.shape
    hid2 = w1.shape[1]
    ntok = bsz * slen
    ng = bsz // _G

    e = embed[jnp.reshape(seq, (-1,))]

    full = lambda shape: pl.BlockSpec(shape, lambda i: (0, 0))
    k_all = pl.pallas_call(
        _encoder_body,
        grid=(ntok // _TT,),
        in_specs=[
            pl.BlockSpec((_TT, hdim), lambda i: (i, 0)),
            full((hdim, hid2)), full((1, hid2)),
            full((hid2, hdim)), full((1, hdim)),
            full((1, hdim)), full((1, hdim)),
            full((hdim, hdim)),
        ],
        out_specs=pl.BlockSpec((_TT, hdim), lambda i: (i, 0)),
        out_shape=jax.ShapeDtypeStruct((ntok, hdim), jnp.float32),
        compiler_params=pltpu.CompilerParams(
            dimension_semantics=("parallel",)),
    )(e, w1, b1.reshape(1, -1), w2, b2.reshape(1, -1),
      ln_g.reshape(1, -1), ln_b.reshape(1, -1), kp_w)

    nc = slen // _C
    ks = k_all.reshape(ng, _G, nc, _C, hdim)
    rr = pl.pallas_call(
        _scan_body,
        grid=(ng, nc),
        in_specs=[
            pl.BlockSpec((1, _G, 1, _C, hdim),
                         lambda g, c: (g, 0, c, 0, 0)),
            pl.BlockSpec((hdim, hdim), lambda g, c: (0, 0)),
            pl.BlockSpec((1, hdim), lambda g, c: (0, 0)),
        ],
        out_specs=pl.BlockSpec((1, _G, hdim), lambda g, c: (g, 0, 0)),
        out_shape=jax.ShapeDtypeStruct((ng, _G, hdim), jnp.float32),
        scratch_shapes=[pltpu.VMEM((hdim, hdim), jnp.float32)
                        for _ in range(_G)],
        compiler_params=pltpu.CompilerParams(
            dimension_semantics=("parallel", "arbitrary")),
    )(ks, rp_w, rp_b.reshape(1, -1))
    rr = rr.reshape(bsz, hdim)

    out = pl.pallas_call(
        _head_body,
        grid=(vocab // _VT,),
        in_specs=[
            pl.BlockSpec((bsz, hdim), lambda i: (0, 0)),
            pl.BlockSpec((hdim, _VT), lambda i: (0, i)),
            pl.BlockSpec((1, _VT), lambda i: (0, i)),
        ],
        out_specs=pl.BlockSpec((bsz, _VT), lambda i: (0, i)),
        out_shape=jax.ShapeDtypeStruct((bsz, vocab), jnp.float32),
        compiler_params=pltpu.CompilerParams(
            dimension_semantics=("parallel",)),
    )(rr, out_w, out_b.reshape(1, -1))
    return out
